# Initial kernel scaffold; baseline (speedup 1.0000x reference)
import functools

import jax
import jax.numpy as jnp
from jax import lax
from jax.experimental import pallas as pl
from jax.experimental.pallas import tpu as pltpu
from jax.experimental.pallas import tpu_sc as plsc

H, T, D, DH = 16, 2048, 32, 64
NC, NS = 2, 16
NW = NC * NS
QPW = (H * T) // NW
QB = 8
NT = QPW // QB
ROWS = QB * D
NEG = jnp.float32(-1e30)
EPS = jnp.float32(1e-9)
SCALE = jnp.float32(0.125)


def _sc_attention(qf, kv, idxf):
    mesh = plsc.VectorSubcoreMesh(
        core_axis_name="c", subcore_axis_name="s", num_cores=NC, num_subcores=NS
    )

    @functools.partial(
        pl.kernel,
        out_type=jax.ShapeDtypeStruct((H * T * DH,), jnp.float32),
        mesh=mesh,
        scratch_types=[
            pltpu.VMEM((ROWS, 128), jnp.float32),
            pltpu.VMEM((ROWS,), jnp.int32),
            pltpu.VMEM((2, 128), jnp.int32),
            pltpu.VMEM((QB * DH,), jnp.float32),
            pltpu.VMEM((QB * DH,), jnp.float32),
            pltpu.VMEM((D, 16), jnp.float32),
            pltpu.VMEM((D,), jnp.float32),
            pltpu.SemaphoreType.DMA,
        ],
    )
    def body(qf_hbm, kv_hbm, idx_hbm, out_hbm,
             kvbuf, idxraw, idxs, qbuf, ybuf, pbuf, wbuf, sem):
        wid = lax.axis_index("s") * NC + lax.axis_index("c")
        head = wid // 2
        h_off = head * T
        qbase = wid * QPW
        i16 = lax.iota(jnp.int32, 16)

        def tile_body(t, carry):
            row0 = qbase + t * QB
            pltpu.sync_copy(idx_hbm.at[pl.ds(row0 * D, ROWS)], idxraw)
            pltpu.sync_copy(qf_hbm.at[pl.ds(row0 * DH, QB * DH)], qbuf)
            for c in range(ROWS // 16):
                iv = idxraw[pl.ds(c * 16, 16)]
                sv = jnp.minimum(jnp.maximum(iv, 0), T - 1) + h_off
                r, cc = divmod(c * 16, 128)
                idxs[r, pl.ds(cc, 16)] = sv
            cp0 = pltpu.async_copy(kv_hbm.at[idxs.at[0]],
                                   kvbuf.at[pl.ds(0, 128)], sem)
            cp1 = pltpu.async_copy(kv_hbm.at[idxs.at[1]],
                                   kvbuf.at[pl.ds(128, 128)], sem)
            cp0.wait()
            cp1.wait()

            def q_body(qi, carry2):
                base = qi * D
                qv = [qbuf[pl.ds(qi * DH + c * 16, 16)] * SCALE
                      for c in range(4)]
                for j in range(D):
                    kr = [kvbuf[base + j, pl.ds(c * 16, 16)] for c in range(4)]
                    p = (qv[0] * kr[0] + qv[1] * kr[1]
                         + qv[2] * kr[2] + qv[3] * kr[3])
                    pbuf[j, :] = p
                sc = []
                for g in range(2):
                    rowg = i16 + (g * 16)
                    acc = jnp.zeros((16,), jnp.float32)
                    for d2 in range(16):
                        colg = jnp.full((16,), d2, jnp.int32)
                        acc = acc + plsc.load_gather(pbuf, [rowg, colg])
                    sc.append(acc)
                pos = row0 + qi - h_off
                i0 = idxraw[pl.ds(base, 16)]
                i1 = idxraw[pl.ds(base + 16, 16)]
                m0 = (i0 >= 0) & (i0 <= pos)
                m1 = (i1 >= 0) & (i1 <= pos)
                s0 = jnp.where(m0, sc[0], NEG)
                s1 = jnp.where(m1, sc[1], NEG)
                mx = jnp.maximum(jnp.max(s0), jnp.max(s1))
                e0 = jnp.where(m0, jnp.exp(s0 - mx), jnp.float32(0.0))
                e1 = jnp.where(m1, jnp.exp(s1 - mx), jnp.float32(0.0))
                den = jnp.maximum(jnp.sum(e0) + jnp.sum(e1), EPS)
                wbuf[pl.ds(0, 16)] = e0 / den
                wbuf[pl.ds(16, 16)] = e1 / den
                acc = [jnp.zeros((16,), jnp.float32) for _ in range(4)]
                for j in range(D):
                    wj = plsc.load_gather(wbuf, [jnp.full((16,), j, jnp.int32)])
                    vr = [kvbuf[base + j, pl.ds(64 + c * 16, 16)]
                          for c in range(4)]
                    acc = [a + wj * r for a, r in zip(acc, vr)]
                for c in range(4):
                    ybuf[pl.ds(qi * DH + c * 16, 16)] = acc[c]
                return carry2

            lax.fori_loop(0, QB, q_body, 0)
            pltpu.sync_copy(ybuf, out_hbm.at[pl.ds(row0 * DH, QB * DH)])
            return carry

        lax.fori_loop(0, NT, tile_body, 0)

    return body(qf, kv, idxf)


def kernel(q, k, v, neigh_idx):
    qf = q[0].reshape(H * T * DH)
    kv = jnp.concatenate([k[0], v[0]], axis=-1).reshape(H * T, 128)
    idxf = neigh_idx.astype(jnp.int32).reshape(H * T * D)
    y = _sc_attention(qf, kv, idxf)
    return y.reshape(1, H, T, DH)


# speedup vs baseline: 50.0373x; 50.0373x over previous
import functools

import jax
import jax.numpy as jnp
from jax import lax
from jax.experimental import pallas as pl
from jax.experimental.pallas import tpu as pltpu
from jax.experimental.pallas import tpu_sc as plsc

H, T, D, DH = 16, 2048, 32, 64
NC, NS = 2, 16
NW = NC * NS
QPW = (H * T) // NW
QB = 8
NT = QPW // QB
ROWS = QB * D
NEG = -1e30
EPS = 1e-9
SCALE = 0.125

_DN = lax.GatherDimensionNumbers(
    offset_dims=(), collapsed_slice_dims=(0,), start_index_map=(0,)
)


def _perm(v, ix):
    return lax.gather(
        v, ix[:, None], _DN, (1,),
        mode=lax.GatherScatterMode.PROMISE_IN_BOUNDS,
    )


def _sc_attention(qf, kv, idxf):
    mesh = plsc.VectorSubcoreMesh(
        core_axis_name="c", subcore_axis_name="s", num_cores=NC, num_subcores=NS
    )

    @functools.partial(
        pl.kernel,
        out_type=jax.ShapeDtypeStruct((H * T * DH,), jnp.float32),
        mesh=mesh,
        scratch_types=[
            pltpu.VMEM((ROWS, 128), jnp.float32),
            pltpu.VMEM((ROWS,), jnp.int32),
            pltpu.VMEM((128,), jnp.int32),
            pltpu.VMEM((128,), jnp.int32),
            pltpu.VMEM((QB * DH,), jnp.float32),
            pltpu.VMEM((QB * DH,), jnp.float32),
            pltpu.SemaphoreType.DMA,
        ],
        compiler_params=pltpu.CompilerParams(needs_layout_passes=False),
    )
    def body(qf_hbm, kv_hbm, idx_hbm, out_hbm,
             kvbuf, idxraw, idxs0, idxs1, qbuf, ybuf, sem):
        wid = lax.axis_index("s") * NC + lax.axis_index("c")
        head = wid // 2
        h_off = head * T
        qbase = wid * QPW
        i16 = lax.iota(jnp.int32, 16)
        bfly = [i16 ^ k for k in (8, 4, 2, 1)]
        lane_splat = [jnp.full((16,), j, jnp.int32) for j in range(16)]
        lane_mask = [i16 == j for j in range(16)]
        zero = jnp.zeros((16,), jnp.float32)

        def lanesum(v):
            for ix in bfly:
                v = v + _perm(v, ix)
            return v

        def lanemax(v):
            for ix in bfly:
                v = jnp.maximum(v, _perm(v, ix))
            return v

        def tile_body(t, carry):
            row0 = qbase + t * QB
            pltpu.sync_copy(idx_hbm.at[pl.ds(row0 * D, ROWS)], idxraw)
            pltpu.sync_copy(qf_hbm.at[pl.ds(row0 * DH, QB * DH)], qbuf)
            for c in range(ROWS // 16):
                iv = idxraw[pl.ds(c * 16, 16)]
                sv = jnp.minimum(jnp.maximum(iv, 0), T - 1) + h_off
                if c < 8:
                    idxs0[pl.ds(c * 16, 16)] = sv
                else:
                    idxs1[pl.ds((c - 8) * 16, 16)] = sv
            cps = [
                pltpu.async_copy(kv_hbm.at[idxs0], kvbuf.at[pl.ds(0, 128)], sem),
                pltpu.async_copy(kv_hbm.at[idxs1], kvbuf.at[pl.ds(128, 128)], sem),
            ]
            for cp in cps:
                cp.wait()

            def q_body(qi, carry2):
                base = qi * D
                qv = [qbuf[pl.ds(qi * DH + c * 16, 16)] * jnp.float32(SCALE)
                      for c in range(4)]
                sc = []
                for g in range(2):
                    parts = []
                    for c0 in (0, 8):
                        js = range(g * 16 + c0, g * 16 + c0 + 8)
                        kr = {j: [kvbuf[base + j, pl.ds(c * 16, 16)]
                                  for c in range(4)] for j in js}
                        p = {j: (qv[0] * kr[j][0] + qv[1] * kr[j][1])
                             + (qv[2] * kr[j][2] + qv[3] * kr[j][3])
                             for j in js}
                        for ix in bfly:
                            p = {j: p[j] + _perm(p[j], ix) for j in js}
                        parts += [jnp.where(lane_mask[j % 16], p[j], zero)
                                  for j in js]
                    while len(parts) > 1:
                        parts = [a + b for a, b in
                                 zip(parts[0::2], parts[1::2])]
                    sc.append(parts[0])
                pos = row0 + qi - h_off
                i0 = idxraw[pl.ds(base, 16)]
                i1 = idxraw[pl.ds(base + 16, 16)]
                m0 = (i0 >= 0) & (i0 <= pos)
                m1 = (i1 >= 0) & (i1 <= pos)
                s0 = jnp.where(m0, sc[0], jnp.float32(NEG))
                s1 = jnp.where(m1, sc[1], jnp.float32(NEG))
                mx = lanemax(jnp.maximum(s0, s1))
                e0 = jnp.where(m0, jnp.exp(s0 - mx), zero)
                e1 = jnp.where(m1, jnp.exp(s1 - mx), zero)
                den = jnp.maximum(lanesum(e0 + e1), jnp.float32(EPS))
                w = [e0 / den, e1 / den]
                acc = [zero] * 8
                for g in range(2):
                    for c0 in (0, 8):
                        js = range(g * 16 + c0, g * 16 + c0 + 8)
                        ws = {j: _perm(w[g], lane_splat[j % 16]) for j in js}
                        for j in js:
                            a = (j // 8) % 2
                            for c in range(4):
                                vr = kvbuf[base + j, pl.ds(64 + c * 16, 16)]
                                acc[a * 4 + c] = acc[a * 4 + c] + ws[j] * vr
                for c in range(4):
                    ybuf[pl.ds(qi * DH + c * 16, 16)] = acc[c] + acc[4 + c]
                return carry2

            lax.fori_loop(0, QB, q_body, 0)
            pltpu.sync_copy(ybuf, out_hbm.at[pl.ds(row0 * DH, QB * DH)])
            return carry

        lax.fori_loop(0, NT, tile_body, 0)

    return body(qf, kv, idxf)


def kernel(q, k, v, neigh_idx):
    qf = q[0].reshape(H * T * DH)
    kv = jnp.concatenate([k[0], v[0]], axis=-1).reshape(H * T, 128)
    idxf = neigh_idx.astype(jnp.int32).reshape(H * T * D)
    y = _sc_attention(qf, kv, idxf)
    return y.reshape(1, H, T, DH)


# speedup vs baseline: 82.1627x; 1.6420x over previous
import functools

import jax
import jax.numpy as jnp
from jax import lax
from jax.experimental import pallas as pl
from jax.experimental.pallas import tpu as pltpu
from jax.experimental.pallas import tpu_sc as plsc

H, T, D, DH = 16, 2048, 32, 64
NC, NS = 2, 16
NW = NC * NS
QPW = (H * T) // NW
QB = 8
NST = 8
SB = QB * NST
NSUP = QPW // SB
ROWS = QB * D
NEG = -1e30
EPS = 1e-9
SCALE = 0.125

_DN = lax.GatherDimensionNumbers(
    offset_dims=(), collapsed_slice_dims=(0,), start_index_map=(0,)
)


def _perm(v, ix):
    return lax.gather(
        v, ix[:, None], _DN, (1,),
        mode=lax.GatherScatterMode.PROMISE_IN_BOUNDS,
    )


def _sc_attention(qf, kv, idxf):
    mesh = plsc.VectorSubcoreMesh(
        core_axis_name="c", subcore_axis_name="s", num_cores=NC, num_subcores=NS
    )

    @functools.partial(
        pl.kernel,
        out_type=jax.ShapeDtypeStruct((H * T * DH,), jnp.float32),
        mesh=mesh,
        scratch_types=[
            pltpu.VMEM((ROWS, 128), jnp.float32),
            pltpu.VMEM((ROWS, 128), jnp.float32),
            pltpu.VMEM((SB * D,), jnp.int32),
            pltpu.VMEM((2 * NST, 128), jnp.int32),
            pltpu.VMEM((SB * DH,), jnp.float32),
            pltpu.VMEM((SB * DH,), jnp.float32),
            pltpu.SemaphoreType.DMA,
            pltpu.SemaphoreType.DMA,
        ],
        compiler_params=pltpu.CompilerParams(needs_layout_passes=False),
    )
    def body(qf_hbm, kv_hbm, idx_hbm, out_hbm,
             kvA, kvB, idxraw, idxs, qbuf, ybuf, semA, semB):
        wid = lax.axis_index("s") * NC + lax.axis_index("c")
        head = wid // 2
        h_off = head * T
        qbase = wid * QPW
        i16 = lax.iota(jnp.int32, 16)
        bfly = [i16 ^ k for k in (8, 4, 2, 1)]
        lane_splat = [jnp.full((16,), j, jnp.int32) for j in range(16)]
        lane_mask = [i16 == j for j in range(16)]
        zero = jnp.zeros((16,), jnp.float32)

        def lanesum(v):
            for ix in bfly:
                v = v + _perm(v, ix)
            return v

        def lanemax(v):
            for ix in bfly:
                v = jnp.maximum(v, _perm(v, ix))
            return v

        bufs = [kvA, kvB]
        sems = [semA, semB]

        def fire(st):
            b = bufs[st % 2]
            s = sems[st % 2]
            return [
                pltpu.async_copy(kv_hbm.at[idxs.at[2 * st]],
                                 b.at[pl.ds(0, 128)], s),
                pltpu.async_copy(kv_hbm.at[idxs.at[2 * st + 1]],
                                 b.at[pl.ds(128, 128)], s),
            ]

        def compute_block(st, row0):
            buf = bufs[st % 2]

            def q_body(qi, carry2):
                base = qi * D
                qoff = st * QB * DH + qi * DH
                qv = [qbuf[pl.ds(qoff + c * 16, 16)] * jnp.float32(SCALE)
                      for c in range(4)]
                sc = []
                for g in range(2):
                    parts = []
                    for c0 in (0, 8):
                        js = range(g * 16 + c0, g * 16 + c0 + 8)
                        kr = {j: [buf[base + j, pl.ds(c * 16, 16)]
                                  for c in range(4)] for j in js}
                        p = {j: (qv[0] * kr[j][0] + qv[1] * kr[j][1])
                             + (qv[2] * kr[j][2] + qv[3] * kr[j][3])
                             for j in js}
                        for ix in bfly:
                            p = {j: p[j] + _perm(p[j], ix) for j in js}
                        parts += [jnp.where(lane_mask[j % 16], p[j], zero)
                                  for j in js]
                    while len(parts) > 1:
                        parts = [a + b for a, b in
                                 zip(parts[0::2], parts[1::2])]
                    sc.append(parts[0])
                pos = row0 + qi - h_off
                ioff = st * QB * D + base
                i0 = idxraw[pl.ds(ioff, 16)]
                i1 = idxraw[pl.ds(ioff + 16, 16)]
                m0 = (i0 >= 0) & (i0 <= pos)
                m1 = (i1 >= 0) & (i1 <= pos)
                s0 = jnp.where(m0, sc[0], jnp.float32(NEG))
                s1 = jnp.where(m1, sc[1], jnp.float32(NEG))
                mx = lanemax(jnp.maximum(s0, s1))
                e0 = jnp.where(m0, jnp.exp(s0 - mx), zero)
                e1 = jnp.where(m1, jnp.exp(s1 - mx), zero)
                den = jnp.maximum(lanesum(e0 + e1), jnp.float32(EPS))
                w = [e0 / den, e1 / den]
                acc = [zero] * 8
                for g in range(2):
                    for c0 in (0, 8):
                        js = range(g * 16 + c0, g * 16 + c0 + 8)
                        ws = {j: _perm(w[g], lane_splat[j % 16]) for j in js}
                        for j in js:
                            a = (j // 8) % 2
                            for c in range(4):
                                vr = buf[base + j, pl.ds(64 + c * 16, 16)]
                                acc[a * 4 + c] = acc[a * 4 + c] + ws[j] * vr
                for c in range(4):
                    ybuf[pl.ds(qoff + c * 16, 16)] = acc[c] + acc[4 + c]
                return carry2

            lax.fori_loop(0, QB, q_body, 0)

        def sup_body(s, carry):
            srow0 = qbase + s * SB
            pltpu.sync_copy(idx_hbm.at[pl.ds(srow0 * D, SB * D)], idxraw)
            pltpu.sync_copy(qf_hbm.at[pl.ds(srow0 * DH, SB * DH)], qbuf)
            for c in range(SB * D // 16):
                iv = idxraw[pl.ds(c * 16, 16)]
                sv = jnp.minimum(jnp.maximum(iv, 0), T - 1) + h_off
                idxs[c // 8, pl.ds((c % 8) * 16, 16)] = sv
            cps = fire(0)
            for st in range(NST):
                nxt = fire(st + 1) if st + 1 < NST else []
                for cp in cps:
                    cp.wait()
                compute_block(st, srow0 + st * QB)
                cps = nxt
            pltpu.sync_copy(ybuf, out_hbm.at[pl.ds(srow0 * DH, SB * DH)])
            return carry

        lax.fori_loop(0, NSUP, sup_body, 0)

    return body(qf, kv, idxf)


def kernel(q, k, v, neigh_idx):
    qf = q[0].reshape(H * T * DH)
    kv = jnp.concatenate([k[0], v[0]], axis=-1).reshape(H * T, 128)
    idxf = neigh_idx.astype(jnp.int32).reshape(H * T * D)
    y = _sc_attention(qf, kv, idxf)
    return y.reshape(1, H, T, DH)


# speedup vs baseline: 84.5666x; 1.0293x over previous
import functools

import jax
import jax.numpy as jnp
from jax import lax
from jax.experimental import pallas as pl
from jax.experimental.pallas import tpu as pltpu
from jax.experimental.pallas import tpu_sc as plsc

H, T, D, DH = 16, 2048, 32, 64
NC, NS = 2, 16
NW = NC * NS
QPW = (H * T) // NW
QB = 8
NST = 8
SB = QB * NST
NSUP = QPW // SB
ROWS = QB * D
NEG = -1e30
EPS = 1e-9
SCALE = 0.125

_DN = lax.GatherDimensionNumbers(
    offset_dims=(), collapsed_slice_dims=(0,), start_index_map=(0,)
)


def _perm(v, ix):
    return lax.gather(
        v, ix[:, None], _DN, (1,),
        mode=lax.GatherScatterMode.PROMISE_IN_BOUNDS,
    )


def _sc_attention(qf, kv, idxf):
    mesh = plsc.VectorSubcoreMesh(
        core_axis_name="c", subcore_axis_name="s", num_cores=NC, num_subcores=NS
    )

    @functools.partial(
        pl.kernel,
        out_type=jax.ShapeDtypeStruct((H * T * DH,), jnp.float32),
        mesh=mesh,
        scratch_types=[
            pltpu.VMEM((ROWS, 128), jnp.float32),
            pltpu.VMEM((ROWS, 128), jnp.float32),
            pltpu.VMEM((SB * D,), jnp.int32),
            pltpu.VMEM((2 * NST, 128), jnp.int32),
            pltpu.VMEM((SB * DH,), jnp.float32),
            pltpu.VMEM((SB * DH,), jnp.float32),
            pltpu.SemaphoreType.DMA,
            pltpu.SemaphoreType.DMA,
        ],
        compiler_params=pltpu.CompilerParams(needs_layout_passes=False),
    )
    def body(qf_hbm, kv_hbm, idx_hbm, out_hbm,
             kvA, kvB, idxraw, idxs, qbuf, ybuf, semA, semB):
        wid = lax.axis_index("s") * NC + lax.axis_index("c")
        head = wid // 2
        h_off = head * T
        qbase = wid * QPW
        i16 = lax.iota(jnp.int32, 16)
        bfly = [i16 ^ k for k in (8, 4, 2, 1)]
        bxor = {k: i16 ^ k for k in (1, 2, 4, 8)}
        selmask = {k: (i16 & k) == 0 for k in (1, 2, 4, 8)}
        lane_splat = [jnp.full((16,), j, jnp.int32) for j in range(16)]
        zero = jnp.zeros((16,), jnp.float32)

        def hadd_tree(vs):
            cur = vs
            for k in (1, 2, 4, 8):
                sel = selmask[k]
                nxt = []
                for a, b in zip(cur[0::2], cur[1::2]):
                    a2 = a + _perm(a, bxor[k])
                    b2 = b + _perm(b, bxor[k])
                    nxt.append(jnp.where(sel, a2, b2))
                cur = nxt
            return cur[0]

        def lanesum(v):
            for ix in bfly:
                v = v + _perm(v, ix)
            return v

        def lanemax(v):
            for ix in bfly:
                v = jnp.maximum(v, _perm(v, ix))
            return v

        bufs = [kvA, kvB]
        sems = [semA, semB]

        def fire(st):
            b = bufs[st % 2]
            s = sems[st % 2]
            return [
                pltpu.async_copy(kv_hbm.at[idxs.at[2 * st]],
                                 b.at[pl.ds(0, 128)], s),
                pltpu.async_copy(kv_hbm.at[idxs.at[2 * st + 1]],
                                 b.at[pl.ds(128, 128)], s),
            ]

        def compute_block(st, row0):
            buf = bufs[st % 2]

            def q_body(qi, carry2):
                base = qi * D
                qoff = st * QB * DH + qi * DH
                qv = [qbuf[pl.ds(qoff + c * 16, 16)] * jnp.float32(SCALE)
                      for c in range(4)]
                sc = []
                for g in range(2):
                    vs = []
                    for c0 in (0, 8):
                        js = range(g * 16 + c0, g * 16 + c0 + 8)
                        kr = {j: [buf[base + j, pl.ds(c * 16, 16)]
                                  for c in range(4)] for j in js}
                        p = {j: (qv[0] * kr[j][0] + qv[1] * kr[j][1])
                             + (qv[2] * kr[j][2] + qv[3] * kr[j][3])
                             for j in js}
                        vs += [p[j] for j in js]
                    sc.append(hadd_tree(vs))
                pos = row0 + qi - h_off
                ioff = st * QB * D + base
                i0 = idxraw[pl.ds(ioff, 16)]
                i1 = idxraw[pl.ds(ioff + 16, 16)]
                m0 = (i0 >= 0) & (i0 <= pos)
                m1 = (i1 >= 0) & (i1 <= pos)
                s0 = jnp.where(m0, sc[0], jnp.float32(NEG))
                s1 = jnp.where(m1, sc[1], jnp.float32(NEG))
                mx = lanemax(jnp.maximum(s0, s1))
                e0 = jnp.where(m0, jnp.exp(s0 - mx), zero)
                e1 = jnp.where(m1, jnp.exp(s1 - mx), zero)
                den = jnp.maximum(lanesum(e0 + e1), jnp.float32(EPS))
                w = [e0 / den, e1 / den]
                acc = [zero] * 8
                for g in range(2):
                    for c0 in (0, 8):
                        js = range(g * 16 + c0, g * 16 + c0 + 8)
                        ws = {j: _perm(w[g], lane_splat[j % 16]) for j in js}
                        for j in js:
                            a = (j // 8) % 2
                            for c in range(4):
                                vr = buf[base + j, pl.ds(64 + c * 16, 16)]
                                acc[a * 4 + c] = acc[a * 4 + c] + ws[j] * vr
                for c in range(4):
                    ybuf[pl.ds(qoff + c * 16, 16)] = acc[c] + acc[4 + c]
                return carry2

            lax.fori_loop(0, QB, q_body, 0)

        def sup_body(s, carry):
            srow0 = qbase + s * SB
            pltpu.sync_copy(idx_hbm.at[pl.ds(srow0 * D, SB * D)], idxraw)
            pltpu.sync_copy(qf_hbm.at[pl.ds(srow0 * DH, SB * DH)], qbuf)
            for c in range(SB * D // 16):
                iv = idxraw[pl.ds(c * 16, 16)]
                sv = jnp.minimum(jnp.maximum(iv, 0), T - 1) + h_off
                idxs[c // 8, pl.ds((c % 8) * 16, 16)] = sv
            cps = fire(0)
            for st in range(NST):
                nxt = fire(st + 1) if st + 1 < NST else []
                for cp in cps:
                    cp.wait()
                compute_block(st, srow0 + st * QB)
                cps = nxt
            pltpu.sync_copy(ybuf, out_hbm.at[pl.ds(srow0 * DH, SB * DH)])
            return carry

        lax.fori_loop(0, NSUP, sup_body, 0)

    return body(qf, kv, idxf)


def kernel(q, k, v, neigh_idx):
    qf = q[0].reshape(H * T * DH)
    kv = jnp.concatenate([k[0], v[0]], axis=-1).reshape(H * T, 128)
    idxf = neigh_idx.astype(jnp.int32).reshape(H * T * D)
    y = _sc_attention(qf, kv, idxf)
    return y.reshape(1, H, T, DH)
